# baseline (speedup 1.0000x reference)
import functools
import math

import jax
import jax.numpy as jnp
from jax import lax
from jax.experimental import pallas as pl
from jax.experimental.pallas import tpu as pltpu
from jax.experimental.pallas import tpu_sc as plsc

_N_CORES = 2
_N_SUBCORES = 16
_N_WORKERS = _N_CORES * _N_SUBCORES


def _make_pack(V, D):
    scale = math.sqrt(float(D))
    n_full = V // 128
    tail = V - n_full * 128
    cpw = (n_full + _N_WORKERS - 1) // _N_WORKERS
    mesh = plsc.VectorSubcoreMesh(core_axis_name="c", subcore_axis_name="s")

    @functools.partial(
        pl.kernel,
        mesh=mesh,
        compiler_params=pltpu.CompilerParams(needs_layout_passes=False),
        out_type=jax.ShapeDtypeStruct((V // 2, 2 * D), jnp.float32),
        scratch_types=[
            pltpu.VMEM((D, 129), jnp.float32),
            pltpu.VMEM((64, 2 * D), jnp.float32),
        ],
    )
    def pack(tt_hbm, packed_hbm, sinp, sout):
        wid = lax.axis_index("s") * _N_CORES + lax.axis_index("c")
        lane16 = jax.lax.broadcasted_iota(jnp.int32, (16,), 0)

        def transpose_col(ct, width):
            def col_body(c, carry):
                q = c // 2
                h2 = c % 2
                cols = jnp.full((16,), c, jnp.int32)
                for d0 in range(0, D, 16):
                    v16 = plsc.load_gather(sinp, [lane16 + d0, cols])
                    sout[q, pl.ds(h2 * D + d0, 16)] = v16 * scale
                return carry

            lax.fori_loop(0, width, col_body, 0, unroll=2)
            pltpu.sync_copy(
                sout.at[pl.ds(0, width // 2), :],
                packed_hbm.at[pl.ds(ct * 64, width // 2), :],
            )

        def chunk_body(i, carry):
            ct = wid * cpw + i

            @pl.when(ct < n_full)
            def _stage_and_do():
                for dt in range(D // 8):
                    pltpu.sync_copy(
                        tt_hbm.at[pl.ds(8 * dt, 8), pl.ds(ct * 128, 128)],
                        sinp.at[pl.ds(8 * dt, 8), pl.ds(0, 128)],
                    )
                transpose_col(ct, 128)

            return carry

        lax.fori_loop(0, cpw, chunk_body, 0)

        if tail:
            @pl.when(wid == _N_WORKERS - 1)
            def _do_tail():
                for d in range(D):
                    pltpu.sync_copy(
                        tt_hbm.at[d, pl.ds(n_full * 128, tail)],
                        sinp.at[d, pl.ds(0, tail)],
                    )
                transpose_col(n_full, tail)

    return pack


def _make_lookup(V, D, H, B):
    mesh = plsc.VectorSubcoreMesh(core_axis_name="c", subcore_axis_name="s")
    n_bt = B // 128
    n_blocks = H * n_bt
    bpw = n_blocks // _N_WORKERS

    @functools.partial(
        pl.kernel,
        mesh=mesh,
        compiler_params=pltpu.CompilerParams(needs_layout_passes=False),
        out_type=jax.ShapeDtypeStruct((H, D, B), jnp.float32),
        scratch_types=[
            pltpu.VMEM((144,), jnp.int32),
            pltpu.VMEM((128,), jnp.int32),
            pltpu.VMEM((128, 2 * D), jnp.float32),
            pltpu.VMEM((D, 129), jnp.float32),
            pltpu.SemaphoreType.DMA,
        ],
    )
    def lookup(xt_hbm, packed_hbm, o3_hbm, idx_v, pidx_v, rows_v, stage, sem):
        wid = lax.axis_index("s") * _N_CORES + lax.axis_index("c")
        lane16 = jax.lax.broadcasted_iota(jnp.int32, (16,), 0)

        def block_body(i, carry):
            blk = wid * bpw + i
            h = blk // n_bt
            bt = blk % n_bt
            pltpu.sync_copy(
                xt_hbm.at[h, pl.ds(bt * 128, 128)], idx_v.at[pl.ds(0, 128)]
            )
            for j in range(8):
                v16 = idx_v[pl.ds(16 * j, 16)]
                pidx_v[pl.ds(16 * j, 16)] = v16 >> 1
            pltpu.async_copy(packed_hbm.at[pidx_v], rows_v, sem).wait()

            def lane_body(l, carry2):
                iv = idx_v[pl.ds(l, 16)]
                off = (iv[0] & 1) * D
                cols = jnp.full((16,), l, jnp.int32)
                for d0 in range(0, D, 16):
                    v16 = rows_v[l, pl.ds(off + d0, 16)]
                    plsc.store_scatter(stage, [lane16 + d0, cols], v16)
                return carry2

            lax.fori_loop(0, 128, lane_body, 0, unroll=2)
            for dt in range(D // 8):
                pltpu.sync_copy(
                    stage.at[pl.ds(8 * dt, 8), pl.ds(0, 128)],
                    o3_hbm.at[h, pl.ds(8 * dt, 8), pl.ds(bt * 128, 128)],
                )
            return carry

        lax.fori_loop(0, bpw, block_body, 0)

    return lookup


def kernel(x, table):
    B, H = x.shape
    V, D = table.shape
    xt = x.T.astype(jnp.int32)
    tt = table.T
    packed = _make_pack(V, D)(tt)
    o3 = _make_lookup(V, D, H, B)(xt, packed)
    return o3.transpose(2, 0, 1)


# speedup vs baseline: 1.4745x; 1.4745x over previous
import functools
import math

import jax
import jax.numpy as jnp
from jax import lax
from jax.experimental import pallas as pl
from jax.experimental.pallas import tpu as pltpu
from jax.experimental.pallas import tpu_sc as plsc

_N_CORES = 2
_N_SUBCORES = 16
_N_WORKERS = _N_CORES * _N_SUBCORES


def _make_pack(V, D):
    scale = math.sqrt(float(D))
    n_full = V // 128
    tail = V - n_full * 128
    cpw = (n_full + _N_WORKERS - 1) // _N_WORKERS
    nv = cpw + (cpw % 2)
    mesh = plsc.VectorSubcoreMesh(core_axis_name="c", subcore_axis_name="s")

    @functools.partial(
        pl.kernel,
        mesh=mesh,
        compiler_params=pltpu.CompilerParams(needs_layout_passes=False),
        out_type=jax.ShapeDtypeStruct((V // 2, 2 * D), jnp.float32),
        scratch_types=[
            pltpu.VMEM((D, 129), jnp.float32),
            pltpu.VMEM((D, 129), jnp.float32),
            pltpu.VMEM((64, 2 * D), jnp.float32),
            pltpu.VMEM((64, 2 * D), jnp.float32),
            pltpu.SemaphoreType.DMA,
            pltpu.SemaphoreType.DMA,
        ],
    )
    def pack(tt_hbm, packed_hbm, si0, si1, so0, so1, sem_in, sem_out):
        wid = lax.axis_index("s") * _N_CORES + lax.axis_index("c")
        lane16 = jax.lax.broadcasted_iota(jnp.int32, (16,), 0)

        def ct_of(i):
            return jnp.minimum(wid * cpw + i, n_full - 1)

        def in_descs(i, sinp):
            ct = ct_of(i)
            return [
                (
                    tt_hbm.at[pl.ds(8 * dt, 8), pl.ds(ct * 128, 128)],
                    sinp.at[pl.ds(8 * dt, 8), pl.ds(0, 128)],
                )
                for dt in range(D // 8)
            ]

        def start_in(i, sinp):
            for s, d in in_descs(i, sinp):
                pltpu.async_copy(s, d, sem_in)

        def wait_in(i, sinp):
            for s, d in in_descs(i, sinp):
                pltpu.make_async_copy(s, d, sem_in).wait()

        def out_desc(i, sout):
            return (sout.at[:, :], packed_hbm.at[pl.ds(ct_of(i) * 64, 64), :])

        def transpose_col(sinp, sout, width):
            def col_body(c, carry):
                q = c // 2
                h2 = c % 2
                cols = jnp.full((16,), c, jnp.int32)
                for d0 in range(0, D, 16):
                    v16 = plsc.load_gather(sinp, [lane16 + d0, cols])
                    sout[q, pl.ds(h2 * D + d0, 16)] = v16 * scale
                return carry

            lax.fori_loop(0, width, col_body, 0, unroll=4)

        def sub_iter(i, sinp, sout, do_wait):
            wait_in(i, sinp)
            if do_wait:
                s2, d2 = out_desc(i, sout)
                pltpu.make_async_copy(s2, d2, sem_out).wait()
            transpose_col(sinp, sout, 128)
            s, d = out_desc(i, sout)
            pltpu.async_copy(s, d, sem_out)
            start_in(i + 2, sinp)

        start_in(0, si0)
        start_in(1, si1)
        sub_iter(0, si0, so0, False)
        sub_iter(1, si1, so1, False)

        def pair_body(k, carry):
            sub_iter(2 * k, si0, so0, True)
            sub_iter(2 * k + 1, si1, so1, True)
            return carry

        lax.fori_loop(1, nv // 2, pair_body, 0)
        wait_in(nv, si0)
        wait_in(nv + 1, si1)
        for sout in (so0, so1):
            s2, d2 = out_desc(0, sout)
            pltpu.make_async_copy(s2, d2, sem_out).wait()

        if tail:
            @pl.when(wid == _N_WORKERS - 1)
            def _do_tail():
                for d in range(D):
                    pltpu.sync_copy(
                        tt_hbm.at[d, pl.ds(n_full * 128, tail)],
                        si0.at[d, pl.ds(0, tail)],
                    )
                transpose_col(si0, so0, tail)
                pltpu.sync_copy(
                    so0.at[pl.ds(0, tail // 2), :],
                    packed_hbm.at[pl.ds(n_full * 64, tail // 2), :],
                )

    return pack


def _make_lookup(V, D, H, B):
    mesh = plsc.VectorSubcoreMesh(core_axis_name="c", subcore_axis_name="s")
    n_bt = B // 128
    n_blocks = H * n_bt
    bpw = n_blocks // _N_WORKERS

    @functools.partial(
        pl.kernel,
        mesh=mesh,
        compiler_params=pltpu.CompilerParams(needs_layout_passes=False),
        out_type=jax.ShapeDtypeStruct((H, D, B), jnp.float32),
        scratch_types=[
            pltpu.VMEM((128,), jnp.int32),
            pltpu.VMEM((128,), jnp.int32),
            pltpu.VMEM((128,), jnp.int32),
            pltpu.VMEM((128,), jnp.int32),
            pltpu.VMEM((144,), jnp.int32),
            pltpu.VMEM((144,), jnp.int32),
            pltpu.VMEM((128, 2 * D), jnp.float32),
            pltpu.VMEM((128, 2 * D), jnp.float32),
            pltpu.VMEM((D, 129), jnp.float32),
            pltpu.VMEM((D, 129), jnp.float32),
            pltpu.SemaphoreType.DMA,
            pltpu.SemaphoreType.DMA,
            pltpu.SemaphoreType.DMA,
        ],
    )
    def lookup(
        xt_hbm, packed_hbm, o3_hbm,
        ix0, ix1, px0, px1, of0, of1, rw0, rw1, st0, st1,
        sem_idx, sem_g, sem_out,
    ):
        wid = lax.axis_index("s") * _N_CORES + lax.axis_index("c")
        lane16 = jax.lax.broadcasted_iota(jnp.int32, (16,), 0)

        def pos_of(i):
            blk = wid * bpw + jnp.minimum(i, bpw - 1)
            return blk // n_bt, blk % n_bt

        def idx_desc(i, ixb):
            h, bt = pos_of(i)
            return (xt_hbm.at[h, pl.ds(bt * 128, 128)], ixb.at[pl.ds(0, 128)])

        def out_descs(i, stb):
            h, bt = pos_of(i)
            return [
                (
                    stb.at[pl.ds(8 * dt, 8), pl.ds(0, 128)],
                    o3_hbm.at[h, pl.ds(8 * dt, 8), pl.ds(bt * 128, 128)],
                )
                for dt in range(D // 8)
            ]

        def compute_pidx(ixb, pxb, ofb):
            for j in range(8):
                v16 = ixb[pl.ds(16 * j, 16)]
                pxb[pl.ds(16 * j, 16)] = v16 >> 1
                ofb[pl.ds(16 * j, 16)] = (v16 & 1) * D

        def transpose_block(ofb, rwb, stb):
            def lane_body(l, carry):
                ov = ofb[pl.ds(l, 16)]
                off = ov[0]
                cols = jnp.full((16,), l, jnp.int32)
                for d0 in range(0, D, 16):
                    v16 = rwb[l, pl.ds(off + d0, 16)]
                    plsc.store_scatter(stb, [lane16 + d0, cols], v16)
                return carry

            lax.fori_loop(0, 128, lane_body, 0, unroll=4)

        def pair(k, do_wait):
            i = 2 * k
            s, d = idx_desc(i, ix0)
            pltpu.make_async_copy(s, d, sem_idx).wait()
            compute_pidx(ix0, px0, of0)
            s, d = idx_desc(i + 1, ix1)
            pltpu.make_async_copy(s, d, sem_idx).wait()
            compute_pidx(ix1, px1, of1)
            g0 = pltpu.async_copy(packed_hbm.at[px0], rw0, sem_g)
            g1 = pltpu.async_copy(packed_hbm.at[px1], rw1, sem_g)
            s, d = idx_desc(i + 2, ix0)
            pltpu.async_copy(s, d, sem_idx)
            s, d = idx_desc(i + 3, ix1)
            pltpu.async_copy(s, d, sem_idx)
            g0.wait()
            if do_wait:
                for s2, d2 in out_descs(i, st0):
                    pltpu.make_async_copy(s2, d2, sem_out).wait()
            transpose_block(of0, rw0, st0)
            for s2, d2 in out_descs(i, st0):
                pltpu.async_copy(s2, d2, sem_out)
            g1.wait()
            if do_wait:
                for s2, d2 in out_descs(i + 1, st1):
                    pltpu.make_async_copy(s2, d2, sem_out).wait()
            transpose_block(of1, rw1, st1)
            for s2, d2 in out_descs(i + 1, st1):
                pltpu.async_copy(s2, d2, sem_out)

        s, d = idx_desc(0, ix0)
        pltpu.async_copy(s, d, sem_idx)
        s, d = idx_desc(1, ix1)
        pltpu.async_copy(s, d, sem_idx)
        pair(0, False)

        def pair_body(k, carry):
            pair(k, True)
            return carry

        lax.fori_loop(1, bpw // 2, pair_body, 0)
        s, d = idx_desc(bpw, ix0)
        pltpu.make_async_copy(s, d, sem_idx).wait()
        s, d = idx_desc(bpw + 1, ix1)
        pltpu.make_async_copy(s, d, sem_idx).wait()
        for stb in (st0, st1):
            for s2, d2 in out_descs(0, stb):
                pltpu.make_async_copy(s2, d2, sem_out).wait()

    return lookup


def kernel(x, table):
    B, H = x.shape
    V, D = table.shape
    xt = x.T.astype(jnp.int32)
    tt = table.T
    packed = _make_pack(V, D)(tt)
    o3 = _make_lookup(V, D, H, B)(xt, packed)
    return o3.transpose(2, 0, 1)


# speedup vs baseline: 2.6905x; 1.8247x over previous
import functools
import math

import jax
import jax.numpy as jnp
from jax import lax
from jax.experimental import pallas as pl
from jax.experimental.pallas import tpu as pltpu
from jax.experimental.pallas import tpu_sc as plsc

_N_CORES = 2
_N_SUBCORES = 16
_N_WORKERS = _N_CORES * _N_SUBCORES


def _make_pack(V, D):
    scale = math.sqrt(float(D))
    n_full = V // 128
    tail = V - n_full * 128
    cpw = (n_full + _N_WORKERS - 1) // _N_WORKERS
    nv = cpw + (cpw % 2)
    mesh = plsc.VectorSubcoreMesh(core_axis_name="c", subcore_axis_name="s")

    @functools.partial(
        pl.kernel,
        mesh=mesh,
        compiler_params=pltpu.CompilerParams(needs_layout_passes=False),
        out_type=jax.ShapeDtypeStruct((V // 2, 2 * D), jnp.float32),
        scratch_types=[
            pltpu.VMEM((D, 129), jnp.float32),
            pltpu.VMEM((D, 129), jnp.float32),
            pltpu.VMEM((64, 2 * D), jnp.float32),
            pltpu.VMEM((64, 2 * D), jnp.float32),
            pltpu.SemaphoreType.DMA,
            pltpu.SemaphoreType.DMA,
        ],
    )
    def pack(tt_hbm, packed_hbm, si0, si1, so0, so1, sem_in, sem_out):
        wid = lax.axis_index("s") * _N_CORES + lax.axis_index("c")
        lane16 = jax.lax.broadcasted_iota(jnp.int32, (16,), 0)

        def ct_of(i):
            return jnp.minimum(wid * cpw + i, n_full - 1)

        def in_descs(i, sinp):
            ct = ct_of(i)
            return [
                (
                    tt_hbm.at[pl.ds(8 * dt, 8), pl.ds(ct * 128, 128)],
                    sinp.at[pl.ds(8 * dt, 8), pl.ds(0, 128)],
                )
                for dt in range(D // 8)
            ]

        def start_in(i, sinp):
            for s, d in in_descs(i, sinp):
                pltpu.async_copy(s, d, sem_in)

        def wait_in(i, sinp):
            for s, d in in_descs(i, sinp):
                pltpu.make_async_copy(s, d, sem_in).wait()

        def out_desc(i, sout):
            return (sout.at[:, :], packed_hbm.at[pl.ds(ct_of(i) * 64, 64), :])

        def transpose_col(sinp, sout, width):
            @plsc.parallel_loop(0, width, 1, unroll=8)
            def col_body(c):
                q = c // 2
                h2 = c % 2
                cols = jnp.full((16,), c, jnp.int32)
                for d0 in range(0, D, 16):
                    v16 = plsc.load_gather(sinp, [lane16 + d0, cols])
                    sout[q, pl.ds(h2 * D + d0, 16)] = v16 * scale

        def sub_iter(i, sinp, sout, do_wait):
            wait_in(i, sinp)
            if do_wait:
                s2, d2 = out_desc(i, sout)
                pltpu.make_async_copy(s2, d2, sem_out).wait()
            transpose_col(sinp, sout, 128)
            s, d = out_desc(i, sout)
            pltpu.async_copy(s, d, sem_out)
            start_in(i + 2, sinp)

        start_in(0, si0)
        start_in(1, si1)
        sub_iter(0, si0, so0, False)
        sub_iter(1, si1, so1, False)

        def pair_body(k, carry):
            sub_iter(2 * k, si0, so0, True)
            sub_iter(2 * k + 1, si1, so1, True)
            return carry

        lax.fori_loop(1, nv // 2, pair_body, 0)
        wait_in(nv, si0)
        wait_in(nv + 1, si1)
        for sout in (so0, so1):
            s2, d2 = out_desc(0, sout)
            pltpu.make_async_copy(s2, d2, sem_out).wait()

        if tail:
            @pl.when(wid == _N_WORKERS - 1)
            def _do_tail():
                for d in range(D):
                    pltpu.sync_copy(
                        tt_hbm.at[d, pl.ds(n_full * 128, tail)],
                        si0.at[d, pl.ds(0, tail)],
                    )
                transpose_col(si0, so0, tail)
                pltpu.sync_copy(
                    so0.at[pl.ds(0, tail // 2), :],
                    packed_hbm.at[pl.ds(n_full * 64, tail // 2), :],
                )

    return pack


def _make_lookup(V, D, H, B):
    mesh = plsc.VectorSubcoreMesh(core_axis_name="c", subcore_axis_name="s")
    n_bt = B // 128
    n_blocks = H * n_bt
    bpw = n_blocks // _N_WORKERS

    @functools.partial(
        pl.kernel,
        mesh=mesh,
        compiler_params=pltpu.CompilerParams(needs_layout_passes=False),
        out_type=jax.ShapeDtypeStruct((H, D, B), jnp.float32),
        scratch_types=[
            pltpu.VMEM((128,), jnp.int32),
            pltpu.VMEM((128,), jnp.int32),
            pltpu.VMEM((128,), jnp.int32),
            pltpu.VMEM((128,), jnp.int32),
            pltpu.VMEM((144,), jnp.int32),
            pltpu.VMEM((144,), jnp.int32),
            pltpu.VMEM((128, 2 * D), jnp.float32),
            pltpu.VMEM((128, 2 * D), jnp.float32),
            pltpu.VMEM((D, 129), jnp.float32),
            pltpu.VMEM((D, 129), jnp.float32),
            pltpu.SemaphoreType.DMA,
            pltpu.SemaphoreType.DMA,
            pltpu.SemaphoreType.DMA,
        ],
    )
    def lookup(
        xt_hbm, packed_hbm, o3_hbm,
        ix0, ix1, px0, px1, of0, of1, rw0, rw1, st0, st1,
        sem_idx, sem_g, sem_out,
    ):
        wid = lax.axis_index("s") * _N_CORES + lax.axis_index("c")
        lane16 = jax.lax.broadcasted_iota(jnp.int32, (16,), 0)

        def pos_of(i):
            blk = wid * bpw + jnp.minimum(i, bpw - 1)
            return blk // n_bt, blk % n_bt

        def idx_desc(i, ixb):
            h, bt = pos_of(i)
            return (xt_hbm.at[h, pl.ds(bt * 128, 128)], ixb.at[pl.ds(0, 128)])

        def out_descs(i, stb):
            h, bt = pos_of(i)
            return [
                (
                    stb.at[pl.ds(8 * dt, 8), pl.ds(0, 128)],
                    o3_hbm.at[h, pl.ds(8 * dt, 8), pl.ds(bt * 128, 128)],
                )
                for dt in range(D // 8)
            ]

        def compute_pidx(ixb, pxb, ofb):
            for j in range(8):
                v16 = ixb[pl.ds(16 * j, 16)]
                pxb[pl.ds(16 * j, 16)] = v16 >> 1
                ofb[pl.ds(16 * j, 16)] = (v16 & 1) * D

        def transpose_block(ofb, rwb, stb):
            @plsc.parallel_loop(0, 128, 1, unroll=8)
            def lane_body(l):
                ov = ofb[pl.ds(l, 16)]
                off = ov[0]
                cols = jnp.full((16,), l, jnp.int32)
                for d0 in range(0, D, 16):
                    v16 = rwb[l, pl.ds(off + d0, 16)]
                    plsc.store_scatter(stb, [lane16 + d0, cols], v16)

        def pair(k, do_wait):
            i = 2 * k
            s, d = idx_desc(i, ix0)
            pltpu.make_async_copy(s, d, sem_idx).wait()
            compute_pidx(ix0, px0, of0)
            s, d = idx_desc(i + 1, ix1)
            pltpu.make_async_copy(s, d, sem_idx).wait()
            compute_pidx(ix1, px1, of1)
            g0 = pltpu.async_copy(packed_hbm.at[px0], rw0, sem_g)
            g1 = pltpu.async_copy(packed_hbm.at[px1], rw1, sem_g)
            s, d = idx_desc(i + 2, ix0)
            pltpu.async_copy(s, d, sem_idx)
            s, d = idx_desc(i + 3, ix1)
            pltpu.async_copy(s, d, sem_idx)
            g0.wait()
            if do_wait:
                for s2, d2 in out_descs(i, st0):
                    pltpu.make_async_copy(s2, d2, sem_out).wait()
            transpose_block(of0, rw0, st0)
            for s2, d2 in out_descs(i, st0):
                pltpu.async_copy(s2, d2, sem_out)
            g1.wait()
            if do_wait:
                for s2, d2 in out_descs(i + 1, st1):
                    pltpu.make_async_copy(s2, d2, sem_out).wait()
            transpose_block(of1, rw1, st1)
            for s2, d2 in out_descs(i + 1, st1):
                pltpu.async_copy(s2, d2, sem_out)

        s, d = idx_desc(0, ix0)
        pltpu.async_copy(s, d, sem_idx)
        s, d = idx_desc(1, ix1)
        pltpu.async_copy(s, d, sem_idx)
        pair(0, False)

        def pair_body(k, carry):
            pair(k, True)
            return carry

        lax.fori_loop(1, bpw // 2, pair_body, 0)
        s, d = idx_desc(bpw, ix0)
        pltpu.make_async_copy(s, d, sem_idx).wait()
        s, d = idx_desc(bpw + 1, ix1)
        pltpu.make_async_copy(s, d, sem_idx).wait()
        for stb in (st0, st1):
            for s2, d2 in out_descs(0, stb):
                pltpu.make_async_copy(s2, d2, sem_out).wait()

    return lookup


def kernel(x, table):
    B, H = x.shape
    V, D = table.shape
    xt = x.T.astype(jnp.int32)
    tt = table.T
    packed = _make_pack(V, D)(tt)
    o3 = _make_lookup(V, D, H, B)(xt, packed)
    return o3.transpose(2, 0, 1)
